# baseline (speedup 1.0000x reference)
import functools

import jax
import jax.numpy as jnp
from jax import lax
from jax.experimental import pallas as pl
from jax.experimental.pallas import tpu as pltpu
from jax.experimental.pallas import tpu_sc as plsc

B = 1024
L = 50
D = 512
T = L + 2

_INFO = plsc.get_sparse_core_info()
_NC = _INFO.num_cores
_NS = _INFO.num_subcores
_NW = _NS
_BPW = B // _NW


def _gather_body(gidx_hbm, didx_hbm, wg_hbm, wd_hbm, eg_hbm, ed_hbm,
                 gidx_v, didx_v, rows_g, rows_d, sem_g, sem_d):
    wid = lax.axis_index("s")
    base = wid * _BPW

    pltpu.sync_copy(gidx_hbm.at[pl.ds(base, _BPW)], gidx_v)
    pltpu.sync_copy(didx_hbm.at[pl.ds(base, _BPW)], didx_v)

    cp_g = pltpu.async_copy(wg_hbm.at[gidx_v], rows_g, sem_g)
    cp_d = pltpu.async_copy(wd_hbm.at[didx_v], rows_d, sem_d)
    cp_g.wait()
    cp_d.wait()
    wr_g = pltpu.async_copy(rows_g, eg_hbm.at[pl.ds(base, _BPW)], sem_g)
    wr_d = pltpu.async_copy(rows_d, ed_hbm.at[pl.ds(base, _BPW)], sem_d)
    wr_g.wait()
    wr_d.wait()


_BB = 16
_NST = B // _BB
_CB = 3


def _assemble_body(f_hbm, eg_hbm, ed_hbm, out_hbm,
                   fbuf, gbuf, dbuf, obuf, sem_f, sem_g, sem_d, sem_o):
    FR = _BB * 200
    ER = _BB * 4
    OR_ = _BB * 208

    def in_cps(i, k):
        return (
            pltpu.make_async_copy(f_hbm.at[pl.ds(i * FR, FR)], fbuf.at[k],
                                  sem_f.at[k]),
            pltpu.make_async_copy(eg_hbm.at[pl.ds(i * ER, ER)], gbuf.at[k],
                                  sem_g.at[k]),
            pltpu.make_async_copy(ed_hbm.at[pl.ds(i * ER, ER)], dbuf.at[k],
                                  sem_d.at[k]),
        )

    def out_cp(i, k):
        return pltpu.make_async_copy(obuf.at[k], out_hbm.at[pl.ds(i * OR_, OR_)],
                                     sem_o.at[k])

    for k in range(_CB):
        for cp in in_cps(k, k):
            cp.start()

    def step(i, _):
        k = lax.rem(i, _CB)
        for cp in in_cps(i, k):
            cp.wait()

        @pl.when(i >= _CB)
        def _():
            out_cp(i - _CB, k).wait()

        for j in range(_BB):
            obuf[k, pl.ds(j * 208, 4)] = gbuf[k, pl.ds(j * 4, 4)]
            obuf[k, pl.ds(j * 208 + 4, 4)] = dbuf[k, pl.ds(j * 4, 4)]
            obuf[k, pl.ds(j * 208 + 8, 200)] = fbuf[k, pl.ds(j * 200, 200)]
        out_cp(i, k).start()

        @pl.when(i + _CB < _NST)
        def _():
            for cp in in_cps(i + _CB, k):
                cp.start()
        return 0

    lax.fori_loop(0, _NST, step, 0)
    for j in range(_NST - _CB, _NST):
        out_cp(j, j % _CB).wait()


@jax.jit
def _run(feature, genre_index, difficulty_index, W_genre, W_difficulty):
    mesh = plsc.VectorSubcoreMesh(core_axis_name="c", subcore_axis_name="s", num_cores=1)
    gather = pl.kernel(
        _gather_body,
        out_type=(jax.ShapeDtypeStruct((B, D), jnp.float32),
                  jax.ShapeDtypeStruct((B, D), jnp.float32)),
        mesh=mesh,
        scratch_types=[
            pltpu.VMEM((_BPW,), jnp.int32),
            pltpu.VMEM((_BPW,), jnp.int32),
            pltpu.VMEM((_BPW, D), jnp.float32),
            pltpu.VMEM((_BPW, D), jnp.float32),
            pltpu.SemaphoreType.DMA,
            pltpu.SemaphoreType.DMA,
        ],
    )
    eg, ed = gather(genre_index, difficulty_index, W_genre, W_difficulty)

    assemble = pl.pallas_call(
        _assemble_body,
        in_specs=[
            pl.BlockSpec(memory_space=pl.ANY),
            pl.BlockSpec(memory_space=pl.ANY),
            pl.BlockSpec(memory_space=pl.ANY),
        ],
        out_specs=pl.BlockSpec(memory_space=pl.ANY),
        out_shape=jax.ShapeDtypeStruct((B * T * 4, 128), jnp.float32),
        scratch_shapes=[
            pltpu.VMEM((_CB, _BB * 200, 128), jnp.float32),
            pltpu.VMEM((_CB, _BB * 4, 128), jnp.float32),
            pltpu.VMEM((_CB, _BB * 4, 128), jnp.float32),
            pltpu.VMEM((_CB, _BB * 208, 128), jnp.float32),
            pltpu.SemaphoreType.DMA((_CB,)),
            pltpu.SemaphoreType.DMA((_CB,)),
            pltpu.SemaphoreType.DMA((_CB,)),
            pltpu.SemaphoreType.DMA((_CB,)),
        ],
    )
    out2 = assemble(feature.reshape(B * L * 4, 128),
                    eg.reshape(B * 4, 128), ed.reshape(B * 4, 128))
    return out2.reshape(B, T, D)


def kernel(feature, genre_index, difficulty_index, W_genre, W_difficulty):
    gidx = genre_index.reshape(B).astype(jnp.int32)
    didx = difficulty_index.reshape(B).astype(jnp.int32)
    return _run(feature, gidx, didx, W_genre, W_difficulty)


# speedup vs baseline: 1.5408x; 1.5408x over previous
import functools

import jax
import jax.numpy as jnp
from jax import lax
from jax.experimental import pallas as pl
from jax.experimental.pallas import tpu as pltpu
from jax.experimental.pallas import tpu_sc as plsc

B = 1024
L = 50
D = 512
T = L + 2

_INFO = plsc.get_sparse_core_info()
_NC = _INFO.num_cores
_NS = _INFO.num_subcores
_NW = _NS
_BPW = B // _NW


def _gather_body(gidx_hbm, didx_hbm, wg_hbm, wd_hbm, eg_hbm, ed_hbm,
                 gidx_v, didx_v, rows_g, rows_d, sem_g, sem_d):
    wid = lax.axis_index("s")
    base = wid * _BPW

    pltpu.sync_copy(gidx_hbm.at[pl.ds(base, _BPW)], gidx_v)
    pltpu.sync_copy(didx_hbm.at[pl.ds(base, _BPW)], didx_v)

    cp_g = pltpu.async_copy(wg_hbm.at[gidx_v], rows_g, sem_g)
    cp_d = pltpu.async_copy(wd_hbm.at[didx_v], rows_d, sem_d)
    cp_g.wait()
    cp_d.wait()
    wr_g = pltpu.async_copy(rows_g, eg_hbm.at[pl.ds(base, _BPW)], sem_g)
    wr_d = pltpu.async_copy(rows_d, ed_hbm.at[pl.ds(base, _BPW)], sem_d)
    wr_g.wait()
    wr_d.wait()


_BB = 32
_NST = B // _BB
_CB = 3


def _assemble_body(f_hbm, eg_hbm, ed_hbm, out_hbm,
                   fbuf, gbuf, dbuf, obuf, sem_f, sem_g, sem_d, sem_o):
    def in_cps(i, k):
        row = i * _BB
        return (
            pltpu.make_async_copy(f_hbm.at[pl.ds(row, _BB)], fbuf.at[k],
                                  sem_f.at[k]),
            pltpu.make_async_copy(eg_hbm.at[pl.ds(row, _BB)], gbuf.at[k],
                                  sem_g.at[k]),
            pltpu.make_async_copy(ed_hbm.at[pl.ds(row, _BB)], dbuf.at[k],
                                  sem_d.at[k]),
        )

    def out_cp(i, k):
        return pltpu.make_async_copy(obuf.at[k], out_hbm.at[pl.ds(i * _BB, _BB)],
                                     sem_o.at[k])

    for k in range(_CB):
        for cp in in_cps(k, k):
            cp.start()

    def step(i, _):
        k = lax.rem(i, _CB)
        for cp in in_cps(i, k):
            cp.wait()

        @pl.when(i >= _CB)
        def _():
            out_cp(i - _CB, k).wait()

        obuf[k, :, 0, :] = gbuf[k]
        obuf[k, :, 1, :] = dbuf[k]
        obuf[k, :, 2:, :] = fbuf[k]
        out_cp(i, k).start()

        @pl.when(i + _CB < _NST)
        def _():
            for cp in in_cps(i + _CB, k):
                cp.start()
        return 0

    lax.fori_loop(0, _NST, step, 0)
    for j in range(_NST - _CB, _NST):
        out_cp(j, j % _CB).wait()


@jax.jit
def _run(feature, genre_index, difficulty_index, W_genre, W_difficulty):
    mesh = plsc.VectorSubcoreMesh(core_axis_name="c", subcore_axis_name="s", num_cores=1)
    gather = pl.kernel(
        _gather_body,
        out_type=(jax.ShapeDtypeStruct((B, D), jnp.float32),
                  jax.ShapeDtypeStruct((B, D), jnp.float32)),
        mesh=mesh,
        scratch_types=[
            pltpu.VMEM((_BPW,), jnp.int32),
            pltpu.VMEM((_BPW,), jnp.int32),
            pltpu.VMEM((_BPW, D), jnp.float32),
            pltpu.VMEM((_BPW, D), jnp.float32),
            pltpu.SemaphoreType.DMA,
            pltpu.SemaphoreType.DMA,
        ],
    )
    eg, ed = gather(genre_index, difficulty_index, W_genre, W_difficulty)

    assemble = pl.pallas_call(
        _assemble_body,
        in_specs=[
            pl.BlockSpec(memory_space=pl.ANY),
            pl.BlockSpec(memory_space=pl.ANY),
            pl.BlockSpec(memory_space=pl.ANY),
        ],
        out_specs=pl.BlockSpec(memory_space=pl.ANY),
        out_shape=jax.ShapeDtypeStruct((B, T, D), jnp.float32),
        scratch_shapes=[
            pltpu.VMEM((_CB, _BB, L, D), jnp.float32),
            pltpu.VMEM((_CB, _BB, D), jnp.float32),
            pltpu.VMEM((_CB, _BB, D), jnp.float32),
            pltpu.VMEM((_CB, _BB, T, D), jnp.float32),
            pltpu.SemaphoreType.DMA((_CB,)),
            pltpu.SemaphoreType.DMA((_CB,)),
            pltpu.SemaphoreType.DMA((_CB,)),
            pltpu.SemaphoreType.DMA((_CB,)),
        ],
    )
    return assemble(feature, eg, ed)


def kernel(feature, genre_index, difficulty_index, W_genre, W_difficulty):
    gidx = genre_index.reshape(B).astype(jnp.int32)
    didx = difficulty_index.reshape(B).astype(jnp.int32)
    return _run(feature, gidx, didx, W_genre, W_difficulty)
